# baseline (device time: 45038 ns/iter reference)
import jax
import jax.numpy as jnp
from jax import lax
from jax.experimental import pallas as pl
from jax.experimental.pallas import tpu as pltpu

N_DEV = 4
N_LAYERS = 3


def _pmod(a):
    return lax.rem(a + N_DEV, N_DEV)


def kernel(x, Win0, Wout0, Win1, Wout1, Win2, Wout2):
    B, D = x.shape
    rp = B // N_DEV

    def body(x_ref, win0_ref, wout0_ref, win1_ref, wout1_ref, win2_ref,
             wout2_ref, out_ref, pbuf0_ref, pslots_ref, agbuf_ref, rsbuf_ref,
             rs_send_sems, rs_recv_sems, ag_send_sems, ag_recv_sems):
        my = lax.axis_index("i")
        wins = [win0_ref, win1_ref, win2_ref]
        wouts = [wout0_ref, wout1_ref, wout2_ref]

        def layer(xv, k):
            h = jnp.maximum(
                jnp.dot(xv, wins[k][:, :], preferred_element_type=jnp.float32),
                0.0,
            )
            return jnp.dot(h, wouts[k][:, :], preferred_element_type=jnp.float32)

        barrier_sem = pltpu.get_barrier_semaphore()
        for d in range(1, N_DEV):
            pl.semaphore_signal(
                barrier_sem, inc=1,
                device_id=(_pmod(my + d),),
                device_id_type=pl.DeviceIdType.MESH,
            )
        pl.semaphore_wait(barrier_sem, N_DEV - 1)

        drain = []

        pbuf0_ref[:, :] = layer(x_ref[:, :], 0)
        rs0 = []
        for d in range(1, N_DEV):
            t = _pmod(my + d)
            r = pltpu.make_async_remote_copy(
                src_ref=pbuf0_ref.at[pl.ds(t * rp, rp), :],
                dst_ref=rsbuf_ref.at[0, d - 1],
                send_sem=rs_send_sems.at[0, d - 1],
                recv_sem=rs_recv_sems.at[0, d - 1],
                device_id=(t,),
                device_id_type=pl.DeviceIdType.MESH,
            )
            r.start()
            rs0.append(r)
        drain += rs0
        for r in rs0:
            r.wait_recv()
        own_x = (
            pbuf0_ref[pl.ds(my * rp, rp), :]
            + rsbuf_ref[0, 0] + rsbuf_ref[0, 1] + rsbuf_ref[0, 2]
        )

        for k in range(N_LAYERS - 1):
            agbuf_ref[k, pl.ds(0, rp), :] = own_x
            ag = []
            for d in range(1, N_DEV):
                t = _pmod(my + d)
                r = pltpu.make_async_remote_copy(
                    src_ref=agbuf_ref.at[k, pl.ds(0, rp), :],
                    dst_ref=agbuf_ref.at[k, pl.ds(d * rp, rp), :],
                    send_sem=ag_send_sems.at[k, d - 1],
                    recv_sem=ag_recv_sems.at[k, d - 1],
                    device_id=(t,),
                    device_id_type=pl.DeviceIdType.MESH,
                )
                r.start()
                ag.append(r)
            drain += ag

            p_own = layer(own_x, k + 1)

            for r in ag:
                r.wait_recv()

            pslots_ref[k, :, :] = layer(agbuf_ref[k, pl.ds(rp, B - rp), :], k + 1)

            rs = []
            for d in range(1, N_DEV):
                t = _pmod(my + d)
                r = pltpu.make_async_remote_copy(
                    src_ref=pslots_ref.at[k, pl.ds((3 - d) * rp, rp), :],
                    dst_ref=rsbuf_ref.at[k + 1, d - 1],
                    send_sem=rs_send_sems.at[k + 1, d - 1],
                    recv_sem=rs_recv_sems.at[k + 1, d - 1],
                    device_id=(t,),
                    device_id_type=pl.DeviceIdType.MESH,
                )
                r.start()
                rs.append(r)
            drain += rs
            for r in rs:
                r.wait_recv()
            own_x = (
                p_own
                + rsbuf_ref[k + 1, 0] + rsbuf_ref[k + 1, 1] + rsbuf_ref[k + 1, 2]
            )

        out_ref[:, :] = own_x

        for r in drain:
            r.wait_send()

    return pl.pallas_call(
        body,
        out_shape=jax.ShapeDtypeStruct((rp, D), jnp.float32),
        in_specs=[pl.BlockSpec(memory_space=pltpu.VMEM)] * 7,
        out_specs=pl.BlockSpec(memory_space=pltpu.VMEM),
        scratch_shapes=[
            pltpu.VMEM((B, D), jnp.float32),
            pltpu.VMEM((2, B - rp, D), jnp.float32),
            pltpu.VMEM((2, B, D), jnp.float32),
            pltpu.VMEM((N_LAYERS, N_DEV - 1, rp, D), jnp.float32),
            pltpu.SemaphoreType.DMA((N_LAYERS, N_DEV - 1)),
            pltpu.SemaphoreType.DMA((N_LAYERS, N_DEV - 1)),
            pltpu.SemaphoreType.DMA((2, N_DEV - 1)),
            pltpu.SemaphoreType.DMA((2, N_DEV - 1)),
        ],
        compiler_params=pltpu.CompilerParams(
            vmem_limit_bytes=100 * 1024 * 1024,
            collective_id=0,
        ),
    )(x, Win0, Wout0, Win1, Wout1, Win2, Wout2)


# device time: 42306 ns/iter; 1.0646x vs baseline; 1.0646x over previous
import jax
import jax.numpy as jnp
from jax import lax
from jax.experimental import pallas as pl
from jax.experimental.pallas import tpu as pltpu

N_DEV = 4
N_LAYERS = 3
N_STREAMS = 2


def _pmod(a):
    return lax.rem(a + N_DEV, N_DEV)


def kernel(x, Win0, Wout0, Win1, Wout1, Win2, Wout2):
    B, D = x.shape
    bs = B // N_STREAMS
    r8 = bs // N_DEV
    r16 = B // N_DEV

    def body(x_ref, win0_ref, wout0_ref, win1_ref, wout1_ref, win2_ref,
             wout2_ref, out_ref, pbuf_ref, p2_ref, xbuf_ref, rsbuf_ref,
             rs2buf_ref, rs_send, rs_recv, ag_send, ag_recv,
             rs2_send, rs2_recv):
        my = lax.axis_index("i")
        wins = [win0_ref, win1_ref, win2_ref]
        wouts = [wout0_ref, wout1_ref, wout2_ref]

        def layer(xv, k):
            h = jnp.maximum(
                jnp.dot(xv, wins[k][:, :], preferred_element_type=jnp.float32),
                0.0,
            )
            return jnp.dot(h, wouts[k][:, :], preferred_element_type=jnp.float32)

        barrier_sem = pltpu.get_barrier_semaphore()
        for d in range(1, N_DEV):
            pl.semaphore_signal(
                barrier_sem, inc=1,
                device_id=(_pmod(my + d),),
                device_id_type=pl.DeviceIdType.MESH,
            )
        pl.semaphore_wait(barrier_sem, N_DEV - 1)

        drain = []

        def fire_rs(l, s):
            rs = []
            for d in range(1, N_DEV):
                t = _pmod(my + d)
                r = pltpu.make_async_remote_copy(
                    src_ref=pbuf_ref.at[l, s, pl.ds(t * r8, r8), :],
                    dst_ref=rsbuf_ref.at[l, s, d - 1],
                    send_sem=rs_send.at[l, s, d - 1],
                    recv_sem=rs_recv.at[l, s, d - 1],
                    device_id=(t,),
                    device_id_type=pl.DeviceIdType.MESH,
                )
                r.start()
                rs.append(r)
            drain.extend(rs)
            return rs

        def reduce_own(l, s):
            return (
                pbuf_ref[l, s, pl.ds(my * r8, r8), :]
                + rsbuf_ref[l, s, 0] + rsbuf_ref[l, s, 1] + rsbuf_ref[l, s, 2]
            )

        def fire_ag(l, s, own):
            xbuf_ref[l, s, pl.ds(my * r8, r8), :] = own
            ag = []
            for d in range(1, N_DEV):
                t = _pmod(my + d)
                r = pltpu.make_async_remote_copy(
                    src_ref=xbuf_ref.at[l, s, pl.ds(my * r8, r8), :],
                    dst_ref=xbuf_ref.at[l, s, pl.ds(my * r8, r8), :],
                    send_sem=ag_send.at[l, s, d - 1],
                    recv_sem=ag_recv.at[l, s, d - 1],
                    device_id=(t,),
                    device_id_type=pl.DeviceIdType.MESH,
                )
                r.start()
                ag.append(r)
            drain.extend(ag)
            return ag

        def wait_all(rdmas):
            for r in rdmas:
                r.wait_recv()

        A, Bs = 0, 1

        pbuf_ref[0, A, :, :] = layer(x_ref[0:bs, :], 0)
        rs_a = fire_rs(0, A)
        pbuf_ref[0, Bs, :, :] = layer(x_ref[bs:B, :], 0)
        rs_b = fire_rs(0, Bs)
        wait_all(rs_a)
        ag_a = fire_ag(0, A, reduce_own(0, A))
        wait_all(rs_b)
        ag_b = fire_ag(0, Bs, reduce_own(0, Bs))

        wait_all(ag_a)
        pbuf_ref[1, A, :, :] = layer(xbuf_ref[0, A, :, :], 1)
        rs_a = fire_rs(1, A)
        wait_all(ag_b)
        pbuf_ref[1, Bs, :, :] = layer(xbuf_ref[0, Bs, :, :], 1)
        rs_b = fire_rs(1, Bs)
        wait_all(rs_a)
        ag_a = fire_ag(1, A, reduce_own(1, A))
        wait_all(rs_b)
        ag_b = fire_ag(1, Bs, reduce_own(1, Bs))

        wait_all(ag_a)
        p2_ref[0:bs, :] = layer(xbuf_ref[1, A, :, :], 2)
        wait_all(ag_b)
        p2_ref[bs:B, :] = layer(xbuf_ref[1, Bs, :, :], 2)
        rs2 = []
        for d in range(1, N_DEV):
            t = _pmod(my + d)
            r = pltpu.make_async_remote_copy(
                src_ref=p2_ref.at[pl.ds(t * r16, r16), :],
                dst_ref=rs2buf_ref.at[d - 1],
                send_sem=rs2_send.at[d - 1],
                recv_sem=rs2_recv.at[d - 1],
                device_id=(t,),
                device_id_type=pl.DeviceIdType.MESH,
            )
            r.start()
            rs2.append(r)
        drain.extend(rs2)
        wait_all(rs2)
        out_ref[:, :] = (
            p2_ref[pl.ds(my * r16, r16), :]
            + rs2buf_ref[0] + rs2buf_ref[1] + rs2buf_ref[2]
        )

        for r in drain:
            r.wait_send()

    return pl.pallas_call(
        body,
        out_shape=jax.ShapeDtypeStruct((r16, D), jnp.float32),
        in_specs=[pl.BlockSpec(memory_space=pltpu.VMEM)] * 7,
        out_specs=pl.BlockSpec(memory_space=pltpu.VMEM),
        scratch_shapes=[
            pltpu.VMEM((2, N_STREAMS, bs, D), jnp.float32),
            pltpu.VMEM((B, D), jnp.float32),
            pltpu.VMEM((2, N_STREAMS, bs, D), jnp.float32),
            pltpu.VMEM((2, N_STREAMS, N_DEV - 1, r8, D), jnp.float32),
            pltpu.VMEM((N_DEV - 1, r16, D), jnp.float32),
            pltpu.SemaphoreType.DMA((2, N_STREAMS, N_DEV - 1)),
            pltpu.SemaphoreType.DMA((2, N_STREAMS, N_DEV - 1)),
            pltpu.SemaphoreType.DMA((2, N_STREAMS, N_DEV - 1)),
            pltpu.SemaphoreType.DMA((2, N_STREAMS, N_DEV - 1)),
            pltpu.SemaphoreType.DMA((N_DEV - 1,)),
            pltpu.SemaphoreType.DMA((N_DEV - 1,)),
        ],
        compiler_params=pltpu.CompilerParams(
            vmem_limit_bytes=100 * 1024 * 1024,
            collective_id=0,
        ),
    )(x, Win0, Wout0, Win1, Wout1, Win2, Wout2)


# device time: 41756 ns/iter; 1.0786x vs baseline; 1.0132x over previous
import jax
import jax.numpy as jnp
from jax import lax
from jax.experimental import pallas as pl
from jax.experimental.pallas import tpu as pltpu

N_DEV = 4
N_LAYERS = 3
N_STREAMS = 2
SEND_ORDER = (2, 1, 3)


def _pmod(a):
    return lax.rem(a + N_DEV, N_DEV)


def kernel(x, Win0, Wout0, Win1, Wout1, Win2, Wout2):
    B, D = x.shape
    bs = B // N_STREAMS
    r8 = bs // N_DEV

    def body(x_ref, win0_ref, wout0_ref, win1_ref, wout1_ref, win2_ref,
             wout2_ref, out_ref, pbuf_ref, xbuf_ref, rsbuf_ref,
             rs_send, rs_recv, ag_send, ag_recv):
        my = lax.axis_index("i")
        wins = [win0_ref, win1_ref, win2_ref]
        wouts = [wout0_ref, wout1_ref, wout2_ref]

        def layer(xv, k):
            h = jnp.maximum(
                jnp.dot(xv, wins[k][:, :], preferred_element_type=jnp.float32),
                0.0,
            )
            return jnp.dot(h, wouts[k][:, :], preferred_element_type=jnp.float32)

        drain = []

        def fire_rs(l, s):
            rs = []
            for d in SEND_ORDER:
                t = _pmod(my + d)
                r = pltpu.make_async_remote_copy(
                    src_ref=pbuf_ref.at[l, s, pl.ds(t * r8, r8), :],
                    dst_ref=rsbuf_ref.at[l, s, d - 1],
                    send_sem=rs_send.at[l, s, d - 1],
                    recv_sem=rs_recv.at[l, s, d - 1],
                    device_id=(t,),
                    device_id_type=pl.DeviceIdType.MESH,
                )
                r.start()
                rs.append(r)
            drain.extend(rs)
            return rs

        def reduce_own(l, s, rs):
            for r in rs:
                r.wait_recv()
            return (
                pbuf_ref[l, s, pl.ds(my * r8, r8), :]
                + rsbuf_ref[l, s, 0] + rsbuf_ref[l, s, 1] + rsbuf_ref[l, s, 2]
            )

        def fire_ag(l, s, own):
            xbuf_ref[l, s, pl.ds(my * r8, r8), :] = own
            ag = []
            for d in SEND_ORDER:
                t = _pmod(my + d)
                r = pltpu.make_async_remote_copy(
                    src_ref=xbuf_ref.at[l, s, pl.ds(my * r8, r8), :],
                    dst_ref=xbuf_ref.at[l, s, pl.ds(my * r8, r8), :],
                    send_sem=ag_send.at[l, s, d - 1],
                    recv_sem=ag_recv.at[l, s, d - 1],
                    device_id=(t,),
                    device_id_type=pl.DeviceIdType.MESH,
                )
                r.start()
                ag.append(r)
            drain.extend(ag)
            return ag

        def wait_all(rdmas):
            for r in rdmas:
                r.wait_recv()

        A, Bs = 0, 1

        xa = jnp.concatenate(
            [x_ref[pl.ds(16 * t, r8), :] for t in range(N_DEV)], axis=0
        )
        pbuf_ref[0, A, :, :] = layer(xa, 0)

        barrier_sem = pltpu.get_barrier_semaphore()
        for d in SEND_ORDER:
            pl.semaphore_signal(
                barrier_sem, inc=1,
                device_id=(_pmod(my + d),),
                device_id_type=pl.DeviceIdType.MESH,
            )
        pl.semaphore_wait(barrier_sem, N_DEV - 1)

        rs_a = fire_rs(0, A)
        xb = jnp.concatenate(
            [x_ref[pl.ds(16 * t + r8, r8), :] for t in range(N_DEV)], axis=0
        )
        pbuf_ref[0, Bs, :, :] = layer(xb, 0)
        rs_b = fire_rs(0, Bs)
        ag_a = fire_ag(0, A, reduce_own(0, A, rs_a))
        ag_b = fire_ag(0, Bs, reduce_own(0, Bs, rs_b))

        wait_all(ag_a)
        pbuf_ref[1, A, :, :] = layer(xbuf_ref[0, A, :, :], 1)
        rs_a = fire_rs(1, A)
        wait_all(ag_b)
        pbuf_ref[1, Bs, :, :] = layer(xbuf_ref[0, Bs, :, :], 1)
        rs_b = fire_rs(1, Bs)
        ag_a = fire_ag(1, A, reduce_own(1, A, rs_a))
        ag_b = fire_ag(1, Bs, reduce_own(1, Bs, rs_b))

        wait_all(ag_a)
        pbuf_ref[2, A, :, :] = layer(xbuf_ref[1, A, :, :], 2)
        rs_a = fire_rs(2, A)
        wait_all(ag_b)
        pbuf_ref[2, Bs, :, :] = layer(xbuf_ref[1, Bs, :, :], 2)
        rs_b = fire_rs(2, Bs)
        out_ref[pl.ds(0, r8), :] = reduce_own(2, A, rs_a)
        out_ref[pl.ds(r8, r8), :] = reduce_own(2, Bs, rs_b)

        for r in drain:
            r.wait_send()

    return pl.pallas_call(
        body,
        out_shape=jax.ShapeDtypeStruct((B // N_DEV, D), jnp.float32),
        in_specs=[pl.BlockSpec(memory_space=pltpu.VMEM)] * 7,
        out_specs=pl.BlockSpec(memory_space=pltpu.VMEM),
        scratch_shapes=[
            pltpu.VMEM((N_LAYERS, N_STREAMS, bs, D), jnp.float32),
            pltpu.VMEM((2, N_STREAMS, bs, D), jnp.float32),
            pltpu.VMEM((N_LAYERS, N_STREAMS, N_DEV - 1, r8, D), jnp.float32),
            pltpu.SemaphoreType.DMA((N_LAYERS, N_STREAMS, N_DEV - 1)),
            pltpu.SemaphoreType.DMA((N_LAYERS, N_STREAMS, N_DEV - 1)),
            pltpu.SemaphoreType.DMA((2, N_STREAMS, N_DEV - 1)),
            pltpu.SemaphoreType.DMA((2, N_STREAMS, N_DEV - 1)),
        ],
        compiler_params=pltpu.CompilerParams(
            vmem_limit_bytes=100 * 1024 * 1024,
            collective_id=0,
        ),
    )(x, Win0, Wout0, Win1, Wout1, Win2, Wout2)


# device time: 31377 ns/iter; 1.4354x vs baseline; 1.3308x over previous
import jax
import jax.numpy as jnp
from jax import lax
from jax.experimental import pallas as pl
from jax.experimental.pallas import tpu as pltpu

N_DEV = 4
N_LAYERS = 3
N_STREAMS = 2
SEND_ORDER = (2, 1, 3)


def _pmod(a):
    return lax.rem(a + N_DEV, N_DEV)


def kernel(x, Win0, Wout0, Win1, Wout1, Win2, Wout2):
    B, D = x.shape
    H = Win0.shape[1]
    bs = B // N_STREAMS
    r8 = bs // N_DEV

    def body(x_ref, win0_ref, wout0_ref, win1_ref, wout1_ref, win2_ref,
             wout2_ref, out_ref, pbuf_ref, xbuf_ref, rsbuf_ref,
             winv_ref, woutv_ref, w_sems, rs_send, rs_recv, ag_send, ag_recv):
        my = lax.axis_index("i")

        w_copies = []
        for k, (wi, wo) in enumerate(
            [(win0_ref, wout0_ref), (win1_ref, wout1_ref), (win2_ref, wout2_ref)]
        ):
            ci = pltpu.make_async_copy(wi, winv_ref.at[k], w_sems.at[2 * k])
            co = pltpu.make_async_copy(wo, woutv_ref.at[k], w_sems.at[2 * k + 1])
            ci.start()
            co.start()
            w_copies.append((ci, co))
        w_waited = set()

        def layer(xv, k):
            if 2 * k not in w_waited:
                w_copies[k][0].wait()
                w_waited.add(2 * k)
            h = jnp.maximum(
                jnp.dot(xv, winv_ref[k], preferred_element_type=jnp.float32),
                0.0,
            )
            if 2 * k + 1 not in w_waited:
                w_copies[k][1].wait()
                w_waited.add(2 * k + 1)
            return jnp.dot(h, woutv_ref[k], preferred_element_type=jnp.float32)

        drain = []

        def fire_rs(l, s):
            rs = []
            for d in SEND_ORDER:
                t = _pmod(my + d)
                r = pltpu.make_async_remote_copy(
                    src_ref=pbuf_ref.at[l, s, pl.ds(t * r8, r8), :],
                    dst_ref=rsbuf_ref.at[l, s, d - 1],
                    send_sem=rs_send.at[l, s, d - 1],
                    recv_sem=rs_recv.at[l, s, d - 1],
                    device_id=(t,),
                    device_id_type=pl.DeviceIdType.MESH,
                )
                r.start()
                rs.append(r)
            drain.extend(rs)
            return rs

        def reduce_own(l, s, rs):
            for r in rs:
                r.wait_recv()
            return (
                pbuf_ref[l, s, pl.ds(my * r8, r8), :]
                + rsbuf_ref[l, s, 0] + rsbuf_ref[l, s, 1] + rsbuf_ref[l, s, 2]
            )

        def fire_ag(l, s, own):
            xbuf_ref[l, s, pl.ds(my * r8, r8), :] = own
            ag = []
            for d in SEND_ORDER:
                t = _pmod(my + d)
                r = pltpu.make_async_remote_copy(
                    src_ref=xbuf_ref.at[l, s, pl.ds(my * r8, r8), :],
                    dst_ref=xbuf_ref.at[l, s, pl.ds(my * r8, r8), :],
                    send_sem=ag_send.at[l, s, d - 1],
                    recv_sem=ag_recv.at[l, s, d - 1],
                    device_id=(t,),
                    device_id_type=pl.DeviceIdType.MESH,
                )
                r.start()
                ag.append(r)
            drain.extend(ag)
            return ag

        def wait_all(rdmas):
            for r in rdmas:
                r.wait_recv()

        A, Bs = 0, 1

        xa = jnp.concatenate(
            [x_ref[pl.ds(16 * t, r8), :] for t in range(N_DEV)], axis=0
        )
        pbuf_ref[0, A, :, :] = layer(xa, 0)

        barrier_sem = pltpu.get_barrier_semaphore()
        for d in SEND_ORDER:
            pl.semaphore_signal(
                barrier_sem, inc=1,
                device_id=(_pmod(my + d),),
                device_id_type=pl.DeviceIdType.MESH,
            )
        pl.semaphore_wait(barrier_sem, N_DEV - 1)

        rs_a = fire_rs(0, A)
        xb = jnp.concatenate(
            [x_ref[pl.ds(16 * t + r8, r8), :] for t in range(N_DEV)], axis=0
        )
        pbuf_ref[0, Bs, :, :] = layer(xb, 0)
        rs_b = fire_rs(0, Bs)
        ag_a = fire_ag(0, A, reduce_own(0, A, rs_a))
        ag_b = fire_ag(0, Bs, reduce_own(0, Bs, rs_b))

        wait_all(ag_a)
        pbuf_ref[1, A, :, :] = layer(xbuf_ref[0, A, :, :], 1)
        rs_a = fire_rs(1, A)
        wait_all(ag_b)
        pbuf_ref[1, Bs, :, :] = layer(xbuf_ref[0, Bs, :, :], 1)
        rs_b = fire_rs(1, Bs)
        ag_a = fire_ag(1, A, reduce_own(1, A, rs_a))
        ag_b = fire_ag(1, Bs, reduce_own(1, Bs, rs_b))

        wait_all(ag_a)
        pbuf_ref[2, A, :, :] = layer(xbuf_ref[1, A, :, :], 2)
        rs_a = fire_rs(2, A)
        wait_all(ag_b)
        pbuf_ref[2, Bs, :, :] = layer(xbuf_ref[1, Bs, :, :], 2)
        rs_b = fire_rs(2, Bs)
        out_ref[pl.ds(0, r8), :] = reduce_own(2, A, rs_a)
        out_ref[pl.ds(r8, r8), :] = reduce_own(2, Bs, rs_b)

        for r in drain:
            r.wait_send()

    return pl.pallas_call(
        body,
        out_shape=jax.ShapeDtypeStruct((B // N_DEV, D), jnp.float32),
        in_specs=(
            [pl.BlockSpec(memory_space=pltpu.VMEM)]
            + [pl.BlockSpec(memory_space=pl.ANY)] * 6
        ),
        out_specs=pl.BlockSpec(memory_space=pltpu.VMEM),
        scratch_shapes=[
            pltpu.VMEM((N_LAYERS, N_STREAMS, bs, D), jnp.float32),
            pltpu.VMEM((2, N_STREAMS, bs, D), jnp.float32),
            pltpu.VMEM((N_LAYERS, N_STREAMS, N_DEV - 1, r8, D), jnp.float32),
            pltpu.VMEM((N_LAYERS, D, H), jnp.float32),
            pltpu.VMEM((N_LAYERS, H, D), jnp.float32),
            pltpu.SemaphoreType.DMA((2 * N_LAYERS,)),
            pltpu.SemaphoreType.DMA((N_LAYERS, N_STREAMS, N_DEV - 1)),
            pltpu.SemaphoreType.DMA((N_LAYERS, N_STREAMS, N_DEV - 1)),
            pltpu.SemaphoreType.DMA((2, N_STREAMS, N_DEV - 1)),
            pltpu.SemaphoreType.DMA((2, N_STREAMS, N_DEV - 1)),
        ],
        compiler_params=pltpu.CompilerParams(
            vmem_limit_bytes=100 * 1024 * 1024,
            collective_id=0,
        ),
    )(x, Win0, Wout0, Win1, Wout1, Win2, Wout2)
